# baseline (speedup 1.0000x reference)
import jax
import jax.numpy as jnp
from jax.experimental import pallas as pl
from jax.experimental.pallas import tpu as pltpu

_BK = 128


def _sdhgcn_body(adj_ref, x_ref, w_ref, out_ref, deg_ref):
    i = pl.program_id(0)
    nblk = pl.num_programs(0)

    a = adj_ref[...].astype(jnp.float32)
    xw = jnp.dot(x_ref[...], w_ref[...],
                 preferred_element_type=jnp.float32)
    part = jax.lax.dot_general(
        a, xw, dimension_numbers=(((0,), (0,)), ((), ())),
        preferred_element_type=jnp.float32)
    dpart = jnp.sum(a, axis=0)

    @pl.when(i == 0)
    def _():
        out_ref[...] = part
        deg_ref[...] = dpart

    @pl.when(i > 0)
    def _():
        out_ref[...] += part
        deg_ref[...] += dpart

    @pl.when(i == nblk - 1)
    def _():
        coeff = jax.lax.rsqrt(jnp.maximum(deg_ref[...], 1.0))
        out_ref[...] = jnp.maximum(out_ref[...] * coeff[:, None], 0.0)


def kernel(X, adj_matrix, weight):
    n, d_in = X.shape
    d_out = weight.shape[1]
    nblk = n // _BK
    return pl.pallas_call(
        _sdhgcn_body,
        grid=(nblk,),
        in_specs=[
            pl.BlockSpec((_BK, n), lambda i: (i, 0)),
            pl.BlockSpec((_BK, d_in), lambda i: (i, 0)),
            pl.BlockSpec((d_in, d_out), lambda i: (0, 0)),
        ],
        out_specs=pl.BlockSpec((n, d_out), lambda i: (0, 0)),
        out_shape=jax.ShapeDtypeStruct((n, d_out), jnp.float32),
        scratch_shapes=[pltpu.VMEM((n,), jnp.float32)],
        compiler_params=pltpu.CompilerParams(
            dimension_semantics=("arbitrary",)),
    )(adj_matrix, X, weight)


# speedup vs baseline: 1.4711x; 1.4711x over previous
import jax
import jax.numpy as jnp
from jax.experimental import pallas as pl
from jax.experimental.pallas import tpu as pltpu

_BC = 256


def _sdhgcn_body(adj_ref, x_ref, w_ref, out_ref, xw_ref):
    @pl.when(pl.program_id(0) == 0)
    def _():
        xw_ref[...] = jnp.dot(x_ref[...], w_ref[...],
                              preferred_element_type=jnp.float32)

    a = adj_ref[...].astype(jnp.float32)
    part = jax.lax.dot_general(
        a, xw_ref[...], dimension_numbers=(((0,), (0,)), ((), ())),
        preferred_element_type=jnp.float32)
    deg = jnp.sum(a, axis=0)
    coeff = jax.lax.rsqrt(jnp.maximum(deg, 1.0))
    out_ref[...] = jnp.maximum(part * coeff[:, None], 0.0)


def kernel(X, adj_matrix, weight):
    n, d_in = X.shape
    d_out = weight.shape[1]
    nblk = n // _BC
    return pl.pallas_call(
        _sdhgcn_body,
        grid=(nblk,),
        in_specs=[
            pl.BlockSpec((n, _BC), lambda i: (0, i)),
            pl.BlockSpec((n, d_in), lambda i: (0, 0)),
            pl.BlockSpec((d_in, d_out), lambda i: (0, 0)),
        ],
        out_specs=pl.BlockSpec((_BC, d_out), lambda i: (i, 0)),
        out_shape=jax.ShapeDtypeStruct((n, d_out), jnp.float32),
        scratch_shapes=[pltpu.VMEM((n, d_out), jnp.float32)],
        compiler_params=pltpu.CompilerParams(
            dimension_semantics=("arbitrary",)),
    )(adj_matrix, X, weight)
